# baseline (speedup 1.0000x reference)
import dataclasses
import functools

import jax
import jax.numpy as jnp
from jax import lax
from jax.experimental import pallas as pl
from jax.experimental.pallas import tpu as pltpu
from jax.experimental.pallas import tpu_sc as plsc

_B = 32
_N = 21824
_NB = 4096
_GRID = -(-_N // _NB)
_M2 = 2 * _N
_LANES = 16
_UNROLL = 4
_NVREG = _M2 // _LANES


def _tc_body(lp_ref, lt_ref, cf_ref, y4_ref, y2_ref, m_ref, stats_ref):
    pid = pl.program_id(0)

    d = lp_ref[...] - lt_ref[...]
    ad = jnp.abs(d)
    sl1 = jnp.where(ad < 1.0, 0.5 * d * d, ad - 0.5)
    l4 = lax.broadcasted_iota(jnp.int32, sl1.shape, 1)
    valid4 = (l4 + pid * (4 * _NB)) < (4 * _N)
    pos4 = valid4 & (y4_ref[...].astype(jnp.int32) > 0)
    loc_r = jnp.sum(jnp.where(pos4, sl1, 0.0), axis=1, keepdims=True)

    xs = cf_ref[...]
    xn = pltpu.roll(xs, 2 * _NB - 1, 1)
    mx = jnp.maximum(xs, xn)
    mn = jnp.minimum(xs, xn)
    lse = mx + jnp.log(1.0 + jnp.exp(mn - mx))
    l2 = lax.broadcasted_iota(jnp.int32, xs.shape, 1)
    even = (l2 % 2) == 0
    valid2 = (l2 + pid * (2 * _NB)) < _M2
    posb = y2_ref[...].astype(jnp.int32) > 0
    ce = lse - jnp.where(posb, xn, xs)
    posm = valid2 & even & posb
    negm = valid2 & even & (~posb)
    m_ref[...] = jnp.where(negm, jnp.maximum(ce, 0.0), 0.0)

    npos_r = jnp.sum(jnp.where(posm, 1.0, 0.0), axis=1, keepdims=True)
    ce_r = jnp.sum(jnp.where(posm, ce, 0.0), axis=1, keepdims=True)

    lane = lax.broadcasted_iota(jnp.int32, (32, 128), 1)
    upd = (jnp.where(lane == 0, npos_r, 0.0)
           + jnp.where(lane == 1, ce_r, 0.0)
           + jnp.where(lane == 2, loc_r, 0.0))

    @pl.when(pid == 0)
    def _():
        stats_ref[...] = jnp.zeros_like(stats_ref)

    stats_ref[...] += upd


def _sc_topk_body(m_hbm, kk_hbm, out_hbm, row_v, k_v, out_v):
    wid = lax.axis_index("s") * 2 + lax.axis_index("c")
    pltpu.sync_copy(m_hbm.at[wid], row_v)
    pltpu.sync_copy(kk_hbm.at[wid], k_v)
    k = jnp.max(k_v[...])

    s0 = jnp.zeros((_LANES,), jnp.float32)
    c0 = jnp.zeros((_LANES,), jnp.int32)

    def pass_a(i, carry):
        s, c = carry
        for u in range(_UNROLL):
            v = row_v[pl.ds((i * _UNROLL + u) * _LANES, _LANES)]
            s = s + v
            c = c + (v > 0.0).astype(jnp.int32)
        return s, c

    svec, cvec = lax.fori_loop(0, _NVREG // _UNROLL, pass_a, (s0, c0))
    sum0 = jnp.sum(svec)
    count0 = jnp.sum(cvec)

    def fast(_):
        return sum0

    def slow(_):
        def bis(_, lohi):
            lo, hi = lohi
            mid = lo + (hi - lo + 1) // 2

            def cb(i, c):
                key = plsc.bitcast(row_v[pl.ds(i * _LANES, _LANES)], jnp.int32)
                return c + (key >= mid).astype(jnp.int32)

            cnt = jnp.sum(lax.fori_loop(0, _NVREG, cb, c0))
            take = cnt >= k
            return (jnp.where(take, mid, lo), jnp.where(take, hi, mid - 1))

        tkey, _hi = lax.fori_loop(0, 31, bis, (jnp.int32(0),
                                               jnp.int32(0x7F800000)))
        tval = jnp.max(plsc.bitcast(jnp.full((_LANES,), tkey, jnp.int32),
                                    jnp.float32))

        def fb(i, carry):
            sg, cg = carry
            v = row_v[pl.ds(i * _LANES, _LANES)]
            gt = plsc.bitcast(v, jnp.int32) > tkey
            return (sg + jnp.where(gt, v, 0.0), cg + gt.astype(jnp.int32))

        sgv, cgv = lax.fori_loop(0, _NVREG, fb, (s0, c0))
        sum_gt = jnp.sum(sgv)
        cnt_gt = jnp.sum(cgv)
        return sum_gt + (k - cnt_gt).astype(jnp.float32) * tval

    res = lax.cond(k >= count0, fast, slow, None)
    out_v[...] = jnp.full((_LANES,), res, jnp.float32)
    pltpu.sync_copy(out_v, out_hbm.at[wid])


def _tc_stage(lp4, lt4, cf2, y4, y2):
    return pl.pallas_call(
        _tc_body,
        grid=(_GRID,),
        in_specs=[
            pl.BlockSpec((32, 4 * _NB), lambda i: (0, i)),
            pl.BlockSpec((32, 4 * _NB), lambda i: (0, i)),
            pl.BlockSpec((32, 2 * _NB), lambda i: (0, i)),
            pl.BlockSpec((32, 4 * _NB), lambda i: (0, i)),
            pl.BlockSpec((32, 2 * _NB), lambda i: (0, i)),
        ],
        out_specs=[
            pl.BlockSpec((32, 2 * _NB), lambda i: (0, i)),
            pl.BlockSpec((32, 128), lambda i: (0, 0)),
        ],
        out_shape=[
            jax.ShapeDtypeStruct((_B, _M2), jnp.float32),
            jax.ShapeDtypeStruct((_B, 128), jnp.float32),
        ],
    )(lp4, lt4, cf2, y4, y2)


def _sc_stage(m, kk):
    mesh = plsc.VectorSubcoreMesh(core_axis_name="c", subcore_axis_name="s")
    cp = pltpu.CompilerParams()
    if "needs_layout_passes" in pltpu.CompilerParams.__dataclass_fields__:
        cp = dataclasses.replace(cp, needs_layout_passes=False)
    fn = pl.kernel(
        _sc_topk_body,
        out_type=jax.ShapeDtypeStruct((_B, _LANES), jnp.float32),
        mesh=mesh,
        compiler_params=cp,
        scratch_types=[
            pltpu.VMEM((_M2,), jnp.float32),
            pltpu.VMEM((_LANES,), jnp.int32),
            pltpu.VMEM((_LANES,), jnp.float32),
        ],
    )
    return fn(m, kk)


def kernel(loc_preds, loc_targets, conf_preds, conf_targets):
    B, N = conf_targets.shape

    lp4 = loc_preds.reshape(B, 4 * N)
    lt4 = loc_targets.reshape(B, 4 * N)
    cf2 = conf_preds.reshape(B, 2 * N)
    y8 = conf_targets.astype(jnp.int8)
    y2 = jnp.repeat(y8, 2, axis=1)
    y4 = jnp.repeat(y8, 4, axis=1)

    m, stats = _tc_stage(lp4, lt4, cf2, y4, y2)

    num_pos = stats[:, 0]
    pos_ce = stats[:, 1]
    loc_s = stats[:, 2]
    num_matched = jnp.sum(num_pos)

    k = jnp.minimum(3 * num_pos.astype(jnp.int32), N - 1)
    kk = jnp.broadcast_to(k[:, None], (B, _LANES))

    topk = _sc_stage(m, kk)

    return (jnp.sum(loc_s) + jnp.sum(pos_ce) + jnp.sum(topk[:, 0])) / num_matched


# speedup vs baseline: 3.6122x; 3.6122x over previous
import dataclasses

import jax
import jax.numpy as jnp
from jax import lax
from jax.experimental import pallas as pl
from jax.experimental.pallas import tpu as pltpu
from jax.experimental.pallas import tpu_sc as plsc

_B = 32
_N = 21824
_NB = 7296
_GRID = -(-_N // _NB)
_LANES = 16
_UNROLL = 4
_NVREG = _N // _LANES


def _conf_body(cp_ref, y_ref, m_ref, stats_ref):
    pid = pl.program_id(0)

    y = y_ref[...]
    pos = y > 0
    x0 = cp_ref[0:32]
    x1 = cp_ref[32:64]
    mx = jnp.maximum(x0, x1)
    mn = jnp.minimum(x0, x1)
    lse = mx + jnp.log(1.0 + jnp.exp(mn - mx))
    ce = lse - jnp.where(pos, x1, x0)

    col = lax.broadcasted_iota(jnp.int32, ce.shape, 1) + pid * _NB
    valid = col < _N
    m_ref[...] = jnp.where(valid & (~pos), jnp.maximum(ce, 0.0), 0.0)

    posm = valid & pos
    npos_r = jnp.sum(jnp.where(posm, 1.0, 0.0), axis=1, keepdims=True)
    ce_r = jnp.sum(jnp.where(posm, ce, 0.0), axis=1, keepdims=True)

    lane = lax.broadcasted_iota(jnp.int32, (32, 128), 1)
    upd = jnp.where(lane == 0, npos_r, 0.0) + jnp.where(lane == 1, ce_r, 0.0)

    @pl.when(pid == 0)
    def _():
        stats_ref[...] = jnp.zeros_like(stats_ref)

    stats_ref[...] += upd


def _loc_body(lp_ref, lt_ref, y_ref, stats_ref):
    pid = pl.program_id(0)

    d = lp_ref[...] - lt_ref[...]
    ad = jnp.abs(d)
    sl1 = jnp.where(ad < 1.0, 0.5 * d * d, ad - 0.5)
    s4 = sl1[0:32] + sl1[32:64] + sl1[64:96] + sl1[96:128]

    col = lax.broadcasted_iota(jnp.int32, s4.shape, 1) + pid * _NB
    posm = (col < _N) & (y_ref[...] > 0)
    loc_r = jnp.sum(jnp.where(posm, s4, 0.0), axis=1, keepdims=True)

    lane = lax.broadcasted_iota(jnp.int32, (32, 128), 1)
    upd = jnp.where(lane == 0, loc_r, 0.0)

    @pl.when(pid == 0)
    def _():
        stats_ref[...] = jnp.zeros_like(stats_ref)

    stats_ref[...] += upd


def _sc_topk_body(m_hbm, kk_hbm, out_hbm, row_v, k_v, out_v):
    wid = lax.axis_index("s") * 2 + lax.axis_index("c")
    pltpu.sync_copy(m_hbm.at[wid], row_v)
    pltpu.sync_copy(kk_hbm.at[wid], k_v)
    k = jnp.max(k_v[...])

    s0 = jnp.zeros((_LANES,), jnp.float32)
    c0 = jnp.zeros((_LANES,), jnp.int32)

    def pass_a(i, carry):
        s, c = carry
        for u in range(_UNROLL):
            v = row_v[pl.ds((i * _UNROLL + u) * _LANES, _LANES)]
            s = s + v
            c = c + (v > 0.0).astype(jnp.int32)
        return s, c

    svec, cvec = lax.fori_loop(0, _NVREG // _UNROLL, pass_a, (s0, c0))
    sum0 = jnp.sum(svec)
    count0 = jnp.sum(cvec)

    def fast(_):
        return sum0

    def slow(_):
        def bis(_, lohi):
            lo, hi = lohi
            mid = lo + (hi - lo + 1) // 2

            def cb(i, c):
                key = plsc.bitcast(row_v[pl.ds(i * _LANES, _LANES)], jnp.int32)
                return c + (key >= mid).astype(jnp.int32)

            cnt = jnp.sum(lax.fori_loop(0, _NVREG, cb, c0))
            take = cnt >= k
            return (jnp.where(take, mid, lo), jnp.where(take, hi, mid - 1))

        tkey, _hi = lax.fori_loop(0, 31, bis, (jnp.int32(0),
                                               jnp.int32(0x7F800000)))
        tval = jnp.max(plsc.bitcast(jnp.full((_LANES,), tkey, jnp.int32),
                                    jnp.float32))

        def fb(i, carry):
            sg, cg = carry
            v = row_v[pl.ds(i * _LANES, _LANES)]
            gt = plsc.bitcast(v, jnp.int32) > tkey
            return (sg + jnp.where(gt, v, 0.0), cg + gt.astype(jnp.int32))

        sgv, cgv = lax.fori_loop(0, _NVREG, fb, (s0, c0))
        sum_gt = jnp.sum(sgv)
        cnt_gt = jnp.sum(cgv)
        return sum_gt + (k - cnt_gt).astype(jnp.float32) * tval

    res = lax.cond(k >= count0, fast, slow, None)
    out_v[...] = jnp.full((_LANES,), res, jnp.float32)
    pltpu.sync_copy(out_v, out_hbm.at[wid])


def _conf_stage(cp_t, y):
    return pl.pallas_call(
        _conf_body,
        grid=(_GRID,),
        in_specs=[
            pl.BlockSpec((64, _NB), lambda i: (0, i)),
            pl.BlockSpec((32, _NB), lambda i: (0, i)),
        ],
        out_specs=[
            pl.BlockSpec((32, _NB), lambda i: (0, i)),
            pl.BlockSpec((32, 128), lambda i: (0, 0)),
        ],
        out_shape=[
            jax.ShapeDtypeStruct((_B, _N), jnp.float32),
            jax.ShapeDtypeStruct((_B, 128), jnp.float32),
        ],
    )(cp_t, y)


def _loc_stage(lp_t, lt_t, y):
    return pl.pallas_call(
        _loc_body,
        grid=(_GRID,),
        in_specs=[
            pl.BlockSpec((128, _NB), lambda i: (0, i)),
            pl.BlockSpec((128, _NB), lambda i: (0, i)),
            pl.BlockSpec((32, _NB), lambda i: (0, i)),
        ],
        out_specs=pl.BlockSpec((32, 128), lambda i: (0, 0)),
        out_shape=jax.ShapeDtypeStruct((_B, 128), jnp.float32),
    )(lp_t, lt_t, y)


def _sc_stage(m, kk):
    mesh = plsc.VectorSubcoreMesh(core_axis_name="c", subcore_axis_name="s")
    cp = pltpu.CompilerParams()
    if "needs_layout_passes" in pltpu.CompilerParams.__dataclass_fields__:
        cp = dataclasses.replace(cp, needs_layout_passes=False)
    fn = pl.kernel(
        _sc_topk_body,
        out_type=jax.ShapeDtypeStruct((_B, _LANES), jnp.float32),
        mesh=mesh,
        compiler_params=cp,
        scratch_types=[
            pltpu.VMEM((_N,), jnp.float32),
            pltpu.VMEM((_LANES,), jnp.int32),
            pltpu.VMEM((_LANES,), jnp.float32),
        ],
    )
    return fn(m, kk)


def kernel(loc_preds, loc_targets, conf_preds, conf_targets):
    B, N = conf_targets.shape

    cp_t = jnp.transpose(conf_preds, (2, 0, 1)).reshape(2 * B, N)
    lp_t = jnp.transpose(loc_preds, (2, 0, 1)).reshape(4 * B, N)
    lt_t = jnp.transpose(loc_targets, (2, 0, 1)).reshape(4 * B, N)
    y = conf_targets.astype(jnp.int32)

    m, stats_c = _conf_stage(cp_t, y)

    num_pos = stats_c[:, 0]
    pos_ce = stats_c[:, 1]
    num_matched = jnp.sum(num_pos)
    k = jnp.minimum(3 * num_pos.astype(jnp.int32), N - 1)
    kk = jnp.broadcast_to(k[:, None], (B, _LANES))

    topk = _sc_stage(m, kk)
    stats_l = _loc_stage(lp_t, lt_t, y)

    return (jnp.sum(stats_l[:, 0]) + jnp.sum(pos_ce)
            + jnp.sum(topk[:, 0])) / num_matched


# speedup vs baseline: 3.7985x; 1.0516x over previous
import dataclasses

import jax
import jax.numpy as jnp
from jax import lax
from jax.experimental import pallas as pl
from jax.experimental.pallas import tpu as pltpu
from jax.experimental.pallas import tpu_sc as plsc

_B = 32
_N = 21824
_NB = 7296
_GRID = -(-_N // _NB)
_LANES = 16
_UNROLL = 4
_NVREG = _N // _LANES


def _conf_body(cp_ref, y_ref, m_ref, stats_ref):
    pid = pl.program_id(0)

    y = y_ref[...]
    pos = y > 0
    x0 = cp_ref[0:32]
    x1 = cp_ref[32:64]
    mx = jnp.maximum(x0, x1)
    mn = jnp.minimum(x0, x1)
    lse = mx + jnp.log(1.0 + jnp.exp(mn - mx))
    ce = lse - jnp.where(pos, x1, x0)

    col = lax.broadcasted_iota(jnp.int32, ce.shape, 1) + pid * _NB
    valid = col < _N
    m_ref[...] = jnp.where(valid & (~pos), jnp.maximum(ce, 0.0), 0.0)

    posm = valid & pos
    npos_r = jnp.sum(jnp.where(posm, 1.0, 0.0), axis=1, keepdims=True)
    ce_r = jnp.sum(jnp.where(posm, ce, 0.0), axis=1, keepdims=True)

    lane = lax.broadcasted_iota(jnp.int32, (32, 128), 1)
    upd = jnp.where(lane == 0, npos_r, 0.0) + jnp.where(lane == 1, ce_r, 0.0)

    @pl.when(pid == 0)
    def _():
        stats_ref[...] = jnp.zeros_like(stats_ref)

    stats_ref[...] += upd


def _loc_body(d_ref, y_ref, stats_ref):
    pid = pl.program_id(0)

    d = d_ref[...]
    ad = jnp.abs(d)
    sl1 = jnp.where(ad < 1.0, 0.5 * d * d, ad - 0.5)
    s4 = sl1[0:32] + sl1[32:64] + sl1[64:96] + sl1[96:128]

    col = lax.broadcasted_iota(jnp.int32, s4.shape, 1) + pid * _NB
    posm = (col < _N) & (y_ref[...] > 0)
    loc_r = jnp.sum(jnp.where(posm, s4, 0.0), axis=1, keepdims=True)

    lane = lax.broadcasted_iota(jnp.int32, (32, 128), 1)
    upd = jnp.where(lane == 0, loc_r, 0.0)

    @pl.when(pid == 0)
    def _():
        stats_ref[...] = jnp.zeros_like(stats_ref)

    stats_ref[...] += upd


def _sc_topk_body(m_hbm, kk_hbm, out_hbm, row_v, k_v, out_v):
    wid = lax.axis_index("s") * 2 + lax.axis_index("c")
    pltpu.sync_copy(m_hbm.at[wid], row_v)
    pltpu.sync_copy(kk_hbm.at[wid], k_v)
    k = jnp.max(k_v[...])

    s0 = jnp.zeros((_LANES,), jnp.float32)
    c0 = jnp.zeros((_LANES,), jnp.int32)

    def pass_a(i, carry):
        s, c = carry
        for u in range(_UNROLL):
            v = row_v[pl.ds((i * _UNROLL + u) * _LANES, _LANES)]
            s = s + v
            c = c + (v > 0.0).astype(jnp.int32)
        return s, c

    svec, cvec = lax.fori_loop(0, _NVREG // _UNROLL, pass_a, (s0, c0))
    sum0 = jnp.sum(svec)
    count0 = jnp.sum(cvec)

    def fast(_):
        return sum0

    def slow(_):
        def bis(_, lohi):
            lo, hi = lohi
            mid = lo + (hi - lo + 1) // 2

            def cb(i, c):
                key = plsc.bitcast(row_v[pl.ds(i * _LANES, _LANES)], jnp.int32)
                return c + (key >= mid).astype(jnp.int32)

            cnt = jnp.sum(lax.fori_loop(0, _NVREG, cb, c0))
            take = cnt >= k
            return (jnp.where(take, mid, lo), jnp.where(take, hi, mid - 1))

        tkey, _hi = lax.fori_loop(0, 31, bis, (jnp.int32(0),
                                               jnp.int32(0x7F800000)))
        tval = jnp.max(plsc.bitcast(jnp.full((_LANES,), tkey, jnp.int32),
                                    jnp.float32))

        def fb(i, carry):
            sg, cg = carry
            v = row_v[pl.ds(i * _LANES, _LANES)]
            gt = plsc.bitcast(v, jnp.int32) > tkey
            return (sg + jnp.where(gt, v, 0.0), cg + gt.astype(jnp.int32))

        sgv, cgv = lax.fori_loop(0, _NVREG, fb, (s0, c0))
        sum_gt = jnp.sum(sgv)
        cnt_gt = jnp.sum(cgv)
        return sum_gt + (k - cnt_gt).astype(jnp.float32) * tval

    res = lax.cond(k >= count0, fast, slow, None)
    out_v[...] = jnp.full((_LANES,), res, jnp.float32)
    pltpu.sync_copy(out_v, out_hbm.at[wid])


def _conf_stage(cp_t, y):
    return pl.pallas_call(
        _conf_body,
        grid=(_GRID,),
        in_specs=[
            pl.BlockSpec((64, _NB), lambda i: (0, i)),
            pl.BlockSpec((32, _NB), lambda i: (0, i)),
        ],
        out_specs=[
            pl.BlockSpec((32, _NB), lambda i: (0, i)),
            pl.BlockSpec((32, 128), lambda i: (0, 0)),
        ],
        out_shape=[
            jax.ShapeDtypeStruct((_B, _N), jnp.float32),
            jax.ShapeDtypeStruct((_B, 128), jnp.float32),
        ],
    )(cp_t, y)


def _loc_stage(d_t, y):
    return pl.pallas_call(
        _loc_body,
        grid=(_GRID,),
        in_specs=[
            pl.BlockSpec((128, _NB), lambda i: (0, i)),
            pl.BlockSpec((32, _NB), lambda i: (0, i)),
        ],
        out_specs=pl.BlockSpec((32, 128), lambda i: (0, 0)),
        out_shape=jax.ShapeDtypeStruct((_B, 128), jnp.float32),
    )(d_t, y)


def _sc_stage(m, kk):
    mesh = plsc.VectorSubcoreMesh(core_axis_name="c", subcore_axis_name="s")
    cp = pltpu.CompilerParams()
    if "needs_layout_passes" in pltpu.CompilerParams.__dataclass_fields__:
        cp = dataclasses.replace(cp, needs_layout_passes=False)
    fn = pl.kernel(
        _sc_topk_body,
        out_type=jax.ShapeDtypeStruct((_B, _LANES), jnp.float32),
        mesh=mesh,
        compiler_params=cp,
        scratch_types=[
            pltpu.VMEM((_N,), jnp.float32),
            pltpu.VMEM((_LANES,), jnp.int32),
            pltpu.VMEM((_LANES,), jnp.float32),
        ],
    )
    return fn(m, kk)


def kernel(loc_preds, loc_targets, conf_preds, conf_targets):
    B, N = conf_targets.shape

    cp_t = jnp.transpose(conf_preds, (2, 0, 1)).reshape(2 * B, N)
    d_t = jnp.transpose(loc_preds - loc_targets, (2, 0, 1)).reshape(4 * B, N)
    y = conf_targets.astype(jnp.int32)

    m, stats_c = _conf_stage(cp_t, y)

    num_pos = stats_c[:, 0]
    pos_ce = stats_c[:, 1]
    num_matched = jnp.sum(num_pos)
    k = jnp.minimum(3 * num_pos.astype(jnp.int32), N - 1)
    kk = jnp.broadcast_to(k[:, None], (B, _LANES))

    topk = _sc_stage(m, kk)
    stats_l = _loc_stage(d_t, y)

    return (jnp.sum(stats_l[:, 0]) + jnp.sum(pos_ce)
            + jnp.sum(topk[:, 0])) / num_matched


# speedup vs baseline: 3.8044x; 1.0016x over previous
import dataclasses

import jax
import jax.numpy as jnp
from jax import lax
from jax.experimental import pallas as pl
from jax.experimental.pallas import tpu as pltpu
from jax.experimental.pallas import tpu_sc as plsc

_B = 32
_N = 21824
_NB = 7296
_GRID = -(-_N // _NB)
_LANES = 16
_UNROLL = 4
_NVREG = _N // _LANES


def _conf_body(cp_ref, y_ref, m_ref, stats_ref):
    pid = pl.program_id(0)

    y = y_ref[...]
    pos = y > 0
    x0 = cp_ref[0:32]
    x1 = cp_ref[32:64]
    mx = jnp.maximum(x0, x1)
    mn = jnp.minimum(x0, x1)
    lse = mx + jnp.log(1.0 + jnp.exp(mn - mx))
    ce = lse - jnp.where(pos, x1, x0)

    col = lax.broadcasted_iota(jnp.int32, ce.shape, 1) + pid * _NB
    valid = col < _N
    m_ref[...] = jnp.where(valid & (~pos), jnp.maximum(ce, 0.0), 0.0)

    posm = valid & pos
    npos_r = jnp.sum(jnp.where(posm, 1.0, 0.0), axis=1, keepdims=True)
    ce_r = jnp.sum(jnp.where(posm, ce, 0.0), axis=1, keepdims=True)

    lane = lax.broadcasted_iota(jnp.int32, (32, 128), 1)
    upd = jnp.where(lane == 0, npos_r, 0.0) + jnp.where(lane == 1, ce_r, 0.0)

    @pl.when(pid == 0)
    def _():
        stats_ref[...] = jnp.zeros_like(stats_ref)

    stats_ref[...] += upd


def _loc_body(d_ref, y_ref, stats_ref):
    pid = pl.program_id(0)

    d = d_ref[...]
    ad = jnp.abs(d)
    sl1 = jnp.where(ad < 1.0, 0.5 * d * d, ad - 0.5)
    s4 = sl1[0:32] + sl1[32:64] + sl1[64:96] + sl1[96:128]

    col = lax.broadcasted_iota(jnp.int32, s4.shape, 1) + pid * _NB
    posm = (col < _N) & (y_ref[...] > 0)
    loc_r = jnp.sum(jnp.where(posm, s4, 0.0), axis=1, keepdims=True)

    lane = lax.broadcasted_iota(jnp.int32, (32, 128), 1)
    upd = jnp.where(lane == 0, loc_r, 0.0)

    @pl.when(pid == 0)
    def _():
        stats_ref[...] = jnp.zeros_like(stats_ref)

    stats_ref[...] += upd


def _sc_topk_body(m_hbm, kk_hbm, out_hbm, row_v, k_v, out_v):
    wid = lax.axis_index("s") * 2 + lax.axis_index("c")
    pltpu.sync_copy(m_hbm.at[wid], row_v)
    pltpu.sync_copy(kk_hbm.at[wid], k_v)
    k = jnp.max(k_v[...])

    s0 = jnp.zeros((_LANES,), jnp.float32)
    c0 = jnp.zeros((_LANES,), jnp.int32)

    def pass_a(i, carry):
        s, c = carry
        for u in range(_UNROLL):
            v = row_v[pl.ds((i * _UNROLL + u) * _LANES, _LANES)]
            s = s + v
            c = c + (v > 0.0).astype(jnp.int32)
        return s, c

    svec, cvec = lax.fori_loop(0, _NVREG // _UNROLL, pass_a, (s0, c0))
    sum0 = jnp.sum(svec)
    count0 = jnp.sum(cvec)

    def fast(_):
        return sum0

    def slow(_):
        def bis(_, lohi):
            lo, hi = lohi
            mid = lo + (hi - lo + 1) // 2

            def cb(i, c):
                key = plsc.bitcast(row_v[pl.ds(i * _LANES, _LANES)], jnp.int32)
                return c + (key >= mid).astype(jnp.int32)

            cnt = jnp.sum(lax.fori_loop(0, _NVREG, cb, c0))
            take = cnt >= k
            return (jnp.where(take, mid, lo), jnp.where(take, hi, mid - 1))

        tkey, _hi = lax.fori_loop(0, 31, bis, (jnp.int32(0),
                                               jnp.int32(0x7F800000)))
        tval = jnp.max(plsc.bitcast(jnp.full((_LANES,), tkey, jnp.int32),
                                    jnp.float32))

        def fb(i, carry):
            sg, cg = carry
            v = row_v[pl.ds(i * _LANES, _LANES)]
            gt = plsc.bitcast(v, jnp.int32) > tkey
            return (sg + jnp.where(gt, v, 0.0), cg + gt.astype(jnp.int32))

        sgv, cgv = lax.fori_loop(0, _NVREG, fb, (s0, c0))
        sum_gt = jnp.sum(sgv)
        cnt_gt = jnp.sum(cgv)
        return sum_gt + (k - cnt_gt).astype(jnp.float32) * tval

    res = lax.cond(k >= count0, fast, slow, None)
    out_v[...] = jnp.full((_LANES,), res, jnp.float32)
    pltpu.sync_copy(out_v, out_hbm.at[wid])


def _conf_stage(cp_t, y):
    return pl.pallas_call(
        _conf_body,
        grid=(_GRID,),
        in_specs=[
            pl.BlockSpec((64, _NB), lambda i: (0, i)),
            pl.BlockSpec((32, _NB), lambda i: (0, i)),
        ],
        out_specs=[
            pl.BlockSpec((32, _NB), lambda i: (0, i)),
            pl.BlockSpec((32, 128), lambda i: (0, 0)),
        ],
        out_shape=[
            jax.ShapeDtypeStruct((_B, _N), jnp.float32),
            jax.ShapeDtypeStruct((_B, 128), jnp.float32),
        ],
    )(cp_t, y)


def _loc_stage(d_t, y):
    return pl.pallas_call(
        _loc_body,
        grid=(_GRID,),
        in_specs=[
            pl.BlockSpec((128, _NB), lambda i: (0, i)),
            pl.BlockSpec((32, _NB), lambda i: (0, i)),
        ],
        out_specs=pl.BlockSpec((32, 128), lambda i: (0, 0)),
        out_shape=jax.ShapeDtypeStruct((_B, 128), jnp.float32),
    )(d_t, y)


def _sc_stage(m, kk):
    mesh = plsc.VectorSubcoreMesh(core_axis_name="c", subcore_axis_name="s")
    cp = pltpu.CompilerParams()
    if "needs_layout_passes" in pltpu.CompilerParams.__dataclass_fields__:
        cp = dataclasses.replace(cp, needs_layout_passes=False)
    fn = pl.kernel(
        _sc_topk_body,
        out_type=jax.ShapeDtypeStruct((_B, _LANES), jnp.float32),
        mesh=mesh,
        compiler_params=cp,
        scratch_types=[
            pltpu.VMEM((_N,), jnp.float32),
            pltpu.VMEM((_LANES,), jnp.int32),
            pltpu.VMEM((_LANES,), jnp.float32),
        ],
    )
    return fn(m, kk)


def kernel(loc_preds, loc_targets, conf_preds, conf_targets):
    B, N = conf_targets.shape

    cp_t = jnp.transpose(conf_preds, (2, 0, 1)).reshape(2 * B, N)
    d_t = jnp.transpose(loc_preds - loc_targets, (2, 0, 1)).reshape(4 * B, N)
    y = conf_targets.astype(jnp.int32)

    m, stats_c = _conf_stage(cp_t, y)

    num_pos = stats_c[:, 0]
    pos_ce = stats_c[:, 1]
    num_matched = jnp.sum(num_pos)
    k = jnp.minimum(3 * num_pos.astype(jnp.int32), N - 1)
    kk = jnp.broadcast_to(k[:, None], (B, _LANES))

    stats_l = _loc_stage(d_t, y)
    topk = _sc_stage(m, kk)

    return (jnp.sum(stats_l[:, 0]) + jnp.sum(pos_ce)
            + jnp.sum(topk[:, 0])) / num_matched
